# Initial kernel scaffold; baseline (speedup 1.0000x reference)
import jax
import jax.numpy as jnp
from jax.experimental import pallas as pl


def kernel(input, h_positions, v_positions):
    raise NotImplementedError("write your pallas kernel here")


# speedup vs baseline: 1.4368x; 1.4368x over previous
import jax
import jax.numpy as jnp
from jax import lax
from jax.experimental import pallas as pl
from jax.experimental.pallas import tpu as pltpu
from jax.experimental.pallas import tpu_sc as plsc

H = W = 384
C = 192
R = 16
NCELL = R * R
NW = 32
RPW = H // NW
HALF = (W * C) // 2
CF = NCELL * C

def _mesh():
    return plsc.VectorSubcoreMesh(core_axis_name="c", subcore_axis_name="s",
                                  num_cores=2, num_subcores=16)


def _sload(ref, i):
    return ref[pl.ds(i, 16)][0]


CHK = CF // 16


def _k1_body(x_hbm, colid_hbm, rowid_hbm, out_hbm,
             xbuf, acc, colid_s, rowid_s, shstage):
    cid = lax.axis_index("c")
    sid = lax.axis_index("s")
    wid = sid * 2 + cid
    base = wid * RPW

    pltpu.sync_copy(colid_hbm, colid_s.at[pl.ds(0, W)])
    pltpu.sync_copy(rowid_hbm, rowid_s.at[pl.ds(0, H)])

    @pl.loop(0, CF // 16)
    def _(q):
        acc[pl.ds(q * 16, 16)] = jnp.zeros((16,), jnp.float32)

    @pl.loop(0, RPW)
    def _(i):
        h = base + i
        r16 = _sload(rowid_s, h) * R
        for half in range(2):
            pltpu.sync_copy(x_hbm.at[h, half], xbuf)

            @pl.loop(0, HALF // C)
            def _(p):
                cb = (r16 + _sload(colid_s, half * (HALF // C) + p)) * C
                for cg in range(C // 16):
                    v = xbuf[pl.ds(p * C + cg * 16, 16)]
                    plsc.addupdate(acc.at[pl.ds(cb + cg * 16, 16)], v)

    HCF = CF // 2
    HCHK = HCF // 16
    for rnd in range(2):
        pltpu.sync_copy(acc.at[pl.ds(rnd * HCF, HCF)], shstage.at[sid])
        plsc.subcore_barrier()

        res = xbuf.at[pl.ds(0, HCHK)]
        tmp = xbuf.at[pl.ds(HCHK, HCHK)]
        pltpu.sync_copy(shstage.at[0, pl.ds(sid * HCHK, HCHK)], res)
        for t in range(1, 16):
            pltpu.sync_copy(shstage.at[t, pl.ds(sid * HCHK, HCHK)], tmp)

            @pl.loop(0, HCHK // 16)
            def _(q):
                res[pl.ds(q * 16, 16)] += tmp[pl.ds(q * 16, 16)]

        pltpu.sync_copy(res,
                        out_hbm.at[cid, pl.ds(rnd * HCF + sid * HCHK, HCHK)])
        plsc.subcore_barrier()


def _k2_body(parts_hbm, recip_hbm, colid_hbm, rowid_hbm, out_hbm,
             means, rowbuf, colid_s, rowid_s):
    cid = lax.axis_index("c")
    sid = lax.axis_index("s")
    wid = sid * 2 + cid
    base = wid * RPW

    pltpu.sync_copy(colid_hbm, colid_s.at[pl.ds(0, W)])
    pltpu.sync_copy(rowid_hbm, rowid_s.at[pl.ds(0, H)])

    CH = CF // 4
    for t in range(4):
        pltpu.sync_copy(parts_hbm.at[0, pl.ds(t * CH, CH)],
                        rowbuf.at[pl.ds(0, CH)])
        pltpu.sync_copy(parts_hbm.at[1, pl.ds(t * CH, CH)],
                        rowbuf.at[pl.ds(CH, CH)])
        pltpu.sync_copy(recip_hbm.at[pl.ds(t * CH, CH)],
                        rowbuf.at[pl.ds(2 * CH, CH)])

        @pl.loop(0, CH // 16)
        def _(q):
            a = rowbuf[pl.ds(q * 16, 16)]
            b = rowbuf[pl.ds(CH + q * 16, 16)]
            rc = rowbuf[pl.ds(2 * CH + q * 16, 16)]
            means[pl.ds(t * CH + q * 16, 16)] = (a + b) * rc

    @pl.loop(0, RPW, init_carry=jnp.int32(-1))
    def _(i, r_prev):
        h = base + i
        r = _sload(rowid_s, h)

        @pl.when(r != r_prev)
        def _():
            @pl.loop(0, W)
            def _(p):
                cb = (r * R + _sload(colid_s, p)) * C
                for cg in range(C // 16):
                    rowbuf[pl.ds(p * C + cg * 16, 16)] = \
                        means[pl.ds(cb + cg * 16, 16)]

        pltpu.sync_copy(rowbuf.at[pl.ds(0, HALF)], out_hbm.at[h, 0])
        pltpu.sync_copy(rowbuf.at[pl.ds(HALF, HALF)], out_hbm.at[h, 1])
        return r


def _grid_pool_sc(x3, col_id, row_id, recip_exp):
    k1 = pl.kernel(
        _k1_body,
        out_type=jax.ShapeDtypeStruct((2, CF), jnp.float32),
        mesh=_mesh(),
        scratch_types=[
            pltpu.VMEM((HALF,), jnp.float32),
            pltpu.VMEM((CF,), jnp.float32),
            pltpu.VMEM((W + 16,), jnp.int32),
            pltpu.VMEM((H + 16,), jnp.int32),
            pltpu.VMEM_SHARED((16, CF // 2), jnp.float32),
        ],
    )
    parts = k1(x3, col_id, row_id)

    k2 = pl.kernel(
        _k2_body,
        out_type=jax.ShapeDtypeStruct((H, 2, HALF), jnp.float32),
        mesh=_mesh(),
        scratch_types=[
            pltpu.VMEM((CF,), jnp.float32),
            pltpu.VMEM((W * C,), jnp.float32),
            pltpu.VMEM((W + 16,), jnp.int32),
            pltpu.VMEM((H + 16,), jnp.int32),
        ],
    )
    return k2(parts, recip_exp, col_id, row_id)


def kernel(input, h_positions, v_positions):
    x3 = input.reshape(H, 2, HALF)

    zero = jnp.zeros((1,), jnp.int32)
    h_bounds = jnp.concatenate([zero, h_positions.astype(jnp.int32),
                                jnp.full((1,), H, jnp.int32)])
    v_bounds = jnp.concatenate([zero, v_positions.astype(jnp.int32),
                                jnp.full((1,), W, jnp.int32)])
    ys = jnp.arange(H, dtype=jnp.int32)
    row_id = jnp.clip(jnp.searchsorted(h_bounds, ys, side="right") - 1,
                      0, R - 1).astype(jnp.int32)
    col_id = jnp.clip(jnp.searchsorted(v_bounds, ys, side="right") - 1,
                      0, R - 1).astype(jnp.int32)
    row_h = (h_bounds[1:] - h_bounds[:-1]).astype(jnp.float32)
    col_w = (v_bounds[1:] - v_bounds[:-1]).astype(jnp.float32)
    area = row_h[:, None] * col_w[None, :]
    recip = 1.0 / jnp.maximum(area, 1.0)
    recip_exp = jnp.repeat(recip.reshape(NCELL), C)

    out = _grid_pool_sc(x3, col_id, row_id, recip_exp)
    return out.reshape(1, H, W, C)


# speedup vs baseline: 2.4264x; 1.6888x over previous
import jax
import jax.numpy as jnp
from jax import lax
from jax.experimental import pallas as pl
from jax.experimental.pallas import tpu as pltpu
from jax.experimental.pallas import tpu_sc as plsc

H = W = 384
C = 192
R = 16
NCELL = R * R
NW = 32
RPW = H // NW
HP = W // 2
CF = NCELL * C
CHK = CF // 16


def _mesh():
    return plsc.VectorSubcoreMesh(core_axis_name="c", subcore_axis_name="s",
                                  num_cores=2, num_subcores=16)


def _sload(ref, i):
    return ref[pl.ds(i, 16)][0]


def _k1_body(x_hbm, colid_hbm, rowid_hbm, out_hbm,
             xbuf, acc, colid_s, rowid_s, shstage):
    cid = lax.axis_index("c")
    sid = lax.axis_index("s")
    wid = sid * 2 + cid
    base = wid * RPW

    pltpu.sync_copy(colid_hbm, colid_s.at[pl.ds(0, W)])
    pltpu.sync_copy(rowid_hbm, rowid_s.at[pl.ds(0, H)])

    @pl.loop(0, CF // 16)
    def _(q):
        acc[pl.ds(q * 16, 16)] = jnp.zeros((16,), jnp.float32)

    @pl.loop(0, RPW)
    def _(i):
        h = base + i
        r16 = _sload(rowid_s, h) * R
        for half in range(2):
            pltpu.sync_copy(x_hbm.at[0, h, pl.ds(half * HP, HP)], xbuf)

            @pl.loop(0, HP)
            def _(p):
                cb = (r16 + _sload(colid_s, half * HP + p)) * C
                for cg in range(C // 16):
                    v = xbuf[p, pl.ds(cg * 16, 16)]
                    plsc.addupdate(acc.at[pl.ds(cb + cg * 16, 16)], v)

    HCF = CF // 2
    HCHK = HCF // 16
    for rnd in range(2):
        pltpu.sync_copy(acc.at[pl.ds(rnd * HCF, HCF)], shstage.at[sid])
        plsc.subcore_barrier()

        res = acc.at[pl.ds(0, HCHK)]
        tmp = acc.at[pl.ds(HCHK, HCHK)]
        pltpu.sync_copy(shstage.at[0, pl.ds(sid * HCHK, HCHK)], res)
        for t in range(1, 16):
            pltpu.sync_copy(shstage.at[t, pl.ds(sid * HCHK, HCHK)], tmp)

            @pl.loop(0, HCHK // 16)
            def _(q):
                res[pl.ds(q * 16, 16)] += tmp[pl.ds(q * 16, 16)]

        pltpu.sync_copy(res,
                        out_hbm.at[cid, pl.ds(rnd * HCF + sid * HCHK, HCHK)])
        plsc.subcore_barrier()


def _k2_body(parts_hbm, recip_hbm, colid_hbm, rowid_hbm, out_hbm,
             means, rowbuf, colid_s, rowid_s, shmeans):
    cid = lax.axis_index("c")
    sid = lax.axis_index("s")
    base = sid * (H // 16)

    pltpu.sync_copy(colid_hbm, colid_s.at[pl.ds(0, W)])
    pltpu.sync_copy(rowid_hbm, rowid_s.at[pl.ds(0, H)])

    off = sid * CHK
    pltpu.sync_copy(parts_hbm.at[0, pl.ds(off, CHK)], means.at[pl.ds(0, CHK)])
    pltpu.sync_copy(parts_hbm.at[1, pl.ds(off, CHK)],
                    means.at[pl.ds(CHK, CHK)])
    pltpu.sync_copy(recip_hbm.at[pl.ds(off, CHK)],
                    means.at[pl.ds(2 * CHK, CHK)])

    @pl.loop(0, CHK // 16)
    def _(q):
        a = means[pl.ds(q * 16, 16)]
        b = means[pl.ds(CHK + q * 16, 16)]
        rc = means[pl.ds(2 * CHK + q * 16, 16)]
        means[pl.ds(q * 16, 16)] = (a + b) * rc

    pltpu.sync_copy(means.at[pl.ds(0, CHK)], shmeans.at[pl.ds(off, CHK)])
    plsc.subcore_barrier()
    pltpu.sync_copy(shmeans, means)

    colbase = cid * HP

    @pl.loop(0, H // 16, init_carry=jnp.int32(-1))
    def _(i, r_prev):
        h = base + i
        r = _sload(rowid_s, h)

        @pl.when(r != r_prev)
        def _():
            @pl.loop(0, HP)
            def _(p):
                cb = (r * R + _sload(colid_s, colbase + p)) * C
                for cg in range(C // 16):
                    rowbuf[p, pl.ds(cg * 16, 16)] = \
                        means[pl.ds(cb + cg * 16, 16)]

        pltpu.sync_copy(rowbuf, out_hbm.at[0, h, pl.ds(colbase, HP)])
        return r


def _grid_pool_sc(x, col_id, row_id, recip_exp):
    k1 = pl.kernel(
        _k1_body,
        out_type=jax.ShapeDtypeStruct((2, CF), jnp.float32),
        mesh=_mesh(),
        scratch_types=[
            pltpu.VMEM((HP, C), jnp.float32),
            pltpu.VMEM((CF,), jnp.float32),
            pltpu.VMEM((W + 16,), jnp.int32),
            pltpu.VMEM((H + 16,), jnp.int32),
            pltpu.VMEM_SHARED((16, CF // 2), jnp.float32),
        ],
    )
    parts = k1(x, col_id, row_id)

    k2 = pl.kernel(
        _k2_body,
        out_type=jax.ShapeDtypeStruct((1, H, W, C), jnp.float32),
        mesh=_mesh(),
        scratch_types=[
            pltpu.VMEM((CF,), jnp.float32),
            pltpu.VMEM((HP, C), jnp.float32),
            pltpu.VMEM((W + 16,), jnp.int32),
            pltpu.VMEM((H + 16,), jnp.int32),
            pltpu.VMEM_SHARED((CF,), jnp.float32),
        ],
    )
    return k2(parts, recip_exp, col_id, row_id)


def kernel(input, h_positions, v_positions):
    hp = h_positions.astype(jnp.int32)
    vp = v_positions.astype(jnp.int32)
    ys = jnp.arange(H, dtype=jnp.int32)
    row_id = (ys[:, None] >= hp[None, :]).sum(axis=1, dtype=jnp.int32)
    col_id = (ys[:, None] >= vp[None, :]).sum(axis=1, dtype=jnp.int32)

    zero = jnp.zeros((1,), jnp.int32)
    h_bounds = jnp.concatenate([zero, hp, jnp.full((1,), H, jnp.int32)])
    v_bounds = jnp.concatenate([zero, vp, jnp.full((1,), W, jnp.int32)])
    row_h = (h_bounds[1:] - h_bounds[:-1]).astype(jnp.float32)
    col_w = (v_bounds[1:] - v_bounds[:-1]).astype(jnp.float32)
    area = row_h[:, None] * col_w[None, :]
    recip = 1.0 / jnp.maximum(area, 1.0)
    recip_exp = jnp.broadcast_to(recip.reshape(NCELL, 1),
                                 (NCELL, C)).reshape(CF)

    return _grid_pool_sc(input, col_id, row_id, recip_exp)


# speedup vs baseline: 6.2073x; 2.5582x over previous
import jax
import jax.numpy as jnp
from jax import lax
from jax.experimental import pallas as pl
from jax.experimental.pallas import tpu as pltpu
from jax.experimental.pallas import tpu_sc as plsc

H = W = 384
C = 192
R = 16
NCELL = R * R
HP = W // 2
CF = NCELL * C
CHK = CF // 16
BH = 64
NG = H // BH


def _mesh():
    return plsc.VectorSubcoreMesh(core_axis_name="c", subcore_axis_name="s",
                                  num_cores=2, num_subcores=16)


def _sload(ref, i):
    return ref[pl.ds(i, 16)][0]


def _k1_body(rowid_smem, xt_ref, mcol_ref, out_ref, g_scr):
    g = pl.program_id(0)

    @pl.when(g == 0)
    def _():
        g_scr[...] = jnp.zeros_like(g_scr)

    def body(h, carry):
        r = rowid_smem[g * BH + h]
        g_scr[r] = g_scr[r] + xt_ref[0, h]
        return carry

    lax.fori_loop(0, BH, body, 0)

    @pl.when(g == NG - 1)
    def _():
        for r in range(R):
            a = jnp.dot(g_scr[r], mcol_ref[...],
                        preferred_element_type=jnp.float32,
                        precision=lax.Precision.HIGHEST)
            out_ref[r] = a.T


def _cell_sums_tc(xt, mcol, row_id):
    return pl.pallas_call(
        _k1_body,
        grid_spec=pltpu.PrefetchScalarGridSpec(
            num_scalar_prefetch=1,
            grid=(NG,),
            in_specs=[
                pl.BlockSpec((1, BH, C, W), lambda g, s: (0, g, 0, 0)),
                pl.BlockSpec((W, R), lambda g, s: (0, 0)),
            ],
            out_specs=pl.BlockSpec((R, R, C), lambda g, s: (0, 0, 0)),
            scratch_shapes=[pltpu.VMEM((R, C, W), jnp.float32)],
        ),
        out_shape=jax.ShapeDtypeStruct((R, R, C), jnp.float32),
    )(row_id, xt, mcol)


def _k2_body(cells_hbm, recip_hbm, colid_hbm, rowid_hbm, out_hbm,
             means, rowbuf, colid_s, rowid_s, shmeans):
    cid = lax.axis_index("c")
    sid = lax.axis_index("s")
    base = sid * (H // 16)

    pltpu.sync_copy(colid_hbm, colid_s.at[pl.ds(0, W)])
    pltpu.sync_copy(rowid_hbm, rowid_s.at[pl.ds(0, H)])

    off = sid * CHK
    pltpu.sync_copy(cells_hbm.at[pl.ds(off, CHK)], means.at[pl.ds(0, CHK)])
    pltpu.sync_copy(recip_hbm.at[pl.ds(off, CHK)],
                    means.at[pl.ds(CHK, CHK)])

    @pl.loop(0, CHK // 16)
    def _(q):
        a = means[pl.ds(q * 16, 16)]
        rc = means[pl.ds(CHK + q * 16, 16)]
        means[pl.ds(q * 16, 16)] = a * rc

    pltpu.sync_copy(means.at[pl.ds(0, CHK)], shmeans.at[pl.ds(off, CHK)])
    plsc.subcore_barrier()
    pltpu.sync_copy(shmeans, means)

    colbase = cid * HP

    @pl.loop(0, H // 16, init_carry=jnp.int32(-1))
    def _(i, r_prev):
        h = base + i
        r = _sload(rowid_s, h)

        @pl.when(r != r_prev)
        def _():
            @pl.loop(0, HP)
            def _(p):
                cb = (r * R + _sload(colid_s, colbase + p)) * C
                for cg in range(C // 16):
                    rowbuf[p, pl.ds(cg * 16, 16)] = \
                        means[pl.ds(cb + cg * 16, 16)]

        pltpu.sync_copy(rowbuf, out_hbm.at[0, h, pl.ds(colbase, HP)])
        return r


def _broadcast_sc(cells, recip_exp, col_id, row_id):
    k2 = pl.kernel(
        _k2_body,
        out_type=jax.ShapeDtypeStruct((1, H, W, C), jnp.float32),
        mesh=_mesh(),
        scratch_types=[
            pltpu.VMEM((CF,), jnp.float32),
            pltpu.VMEM((HP, C), jnp.float32),
            pltpu.VMEM((W + 16,), jnp.int32),
            pltpu.VMEM((H + 16,), jnp.int32),
            pltpu.VMEM_SHARED((CF,), jnp.float32),
        ],
    )
    return k2(cells, recip_exp, col_id, row_id)


def kernel(input, h_positions, v_positions):
    hp = h_positions.astype(jnp.int32)
    vp = v_positions.astype(jnp.int32)
    ys = jnp.arange(H, dtype=jnp.int32)
    row_id = (ys[:, None] >= hp[None, :]).sum(axis=1, dtype=jnp.int32)
    col_id = (ys[:, None] >= vp[None, :]).sum(axis=1, dtype=jnp.int32)
    mcol = (col_id[:, None] ==
            jnp.arange(R, dtype=jnp.int32)[None, :]).astype(jnp.float32)

    zero = jnp.zeros((1,), jnp.int32)
    h_bounds = jnp.concatenate([zero, hp, jnp.full((1,), H, jnp.int32)])
    v_bounds = jnp.concatenate([zero, vp, jnp.full((1,), W, jnp.int32)])
    row_h = (h_bounds[1:] - h_bounds[:-1]).astype(jnp.float32)
    col_w = (v_bounds[1:] - v_bounds[:-1]).astype(jnp.float32)
    area = row_h[:, None] * col_w[None, :]
    recip = 1.0 / jnp.maximum(area, 1.0)
    recip_exp = jnp.broadcast_to(recip.reshape(NCELL, 1),
                                 (NCELL, C)).reshape(CF)

    xt = jnp.transpose(input, (0, 1, 3, 2))
    cells = _cell_sums_tc(xt, mcol, row_id).reshape(CF)
    return _broadcast_sc(cells, recip_exp, col_id, row_id)


# speedup vs baseline: 13.7389x; 2.2134x over previous
import jax
import jax.numpy as jnp
from jax import lax
from jax.experimental import pallas as pl
from jax.experimental.pallas import tpu as pltpu
from jax.experimental.pallas import tpu_sc as plsc

H = W = 384
C = 192
R = 16
NW = 32
RPW = H // NW
BH = 64
NG = H // BH


def _mesh():
    return plsc.VectorSubcoreMesh(core_axis_name="c", subcore_axis_name="s",
                                  num_cores=2, num_subcores=16)


def _sload(ref, i):
    return ref[pl.ds(i, 16)][0]


def _k1_body(rowid_smem, xt_ref, mcol_ref, mcolt_ref, recip_ref, out_ref,
             g_scr):
    g = pl.program_id(0)

    @pl.when(g == 0)
    def _():
        g_scr[...] = jnp.zeros_like(g_scr)

    def body(h, carry):
        r = rowid_smem[g * BH + h]
        g_scr[r] = g_scr[r] + xt_ref[0, h]
        return carry

    lax.fori_loop(0, BH, body, 0)

    @pl.when(g == NG - 1)
    def _():
        for r in range(R):
            a = jnp.dot(g_scr[r], mcol_ref[...],
                        preferred_element_type=jnp.float32,
                        precision=lax.Precision.HIGHEST)
            m = a * recip_ref[r].reshape(1, R)
            out_ref[r] = jnp.dot(m, mcolt_ref[...],
                                 preferred_element_type=jnp.float32,
                                 precision=lax.Precision.HIGHEST)


def _expanded_rows_tc(xt, mcol, mcolt, recip, row_id):
    return pl.pallas_call(
        _k1_body,
        grid_spec=pltpu.PrefetchScalarGridSpec(
            num_scalar_prefetch=1,
            grid=(NG,),
            in_specs=[
                pl.BlockSpec((1, BH, C, W), lambda g, s: (0, g, 0, 0)),
                pl.BlockSpec((W, R), lambda g, s: (0, 0)),
                pl.BlockSpec((R, W), lambda g, s: (0, 0)),
                pl.BlockSpec((R, R), lambda g, s: (0, 0)),
            ],
            out_specs=pl.BlockSpec((R, C, W), lambda g, s: (0, 0, 0)),
            scratch_shapes=[pltpu.VMEM((R, C, W), jnp.float32)],
        ),
        out_shape=jax.ShapeDtypeStruct((R, C, W), jnp.float32),
    )(row_id, xt, mcol, mcolt, recip)


def _k2_body(mexp_hbm, rowid_hbm, out_hbm, rowbuf, rowid_s):
    cid = lax.axis_index("c")
    sid = lax.axis_index("s")
    wid = sid * 2 + cid
    base = wid * RPW

    pltpu.sync_copy(rowid_hbm, rowid_s.at[pl.ds(0, H)])

    @pl.loop(0, RPW, init_carry=jnp.int32(-1))
    def _(i, r_prev):
        h = base + i
        r = _sload(rowid_s, h)

        @pl.when(r != r_prev)
        def _():
            pltpu.sync_copy(mexp_hbm.at[r], rowbuf)

        pltpu.sync_copy(rowbuf, out_hbm.at[0, h])
        return r


def _broadcast_sc(mexp, row_id):
    k2 = pl.kernel(
        _k2_body,
        out_type=jax.ShapeDtypeStruct((1, H, C, W), jnp.float32),
        mesh=_mesh(),
        scratch_types=[
            pltpu.VMEM((C, W), jnp.float32),
            pltpu.VMEM((H + 16,), jnp.int32),
        ],
    )
    return k2(mexp, row_id)


def kernel(input, h_positions, v_positions):
    hp = h_positions.astype(jnp.int32)
    vp = v_positions.astype(jnp.int32)
    ys = jnp.arange(H, dtype=jnp.int32)
    row_id = (ys[:, None] >= hp[None, :]).sum(axis=1, dtype=jnp.int32)
    col_id = (ys[:, None] >= vp[None, :]).sum(axis=1, dtype=jnp.int32)
    segs = jnp.arange(R, dtype=jnp.int32)
    mcol = (col_id[:, None] == segs[None, :]).astype(jnp.float32)
    mcolt = (col_id[None, :] == segs[:, None]).astype(jnp.float32)

    zero = jnp.zeros((1,), jnp.int32)
    h_bounds = jnp.concatenate([zero, hp, jnp.full((1,), H, jnp.int32)])
    v_bounds = jnp.concatenate([zero, vp, jnp.full((1,), W, jnp.int32)])
    row_h = (h_bounds[1:] - h_bounds[:-1]).astype(jnp.float32)
    col_w = (v_bounds[1:] - v_bounds[:-1]).astype(jnp.float32)
    area = row_h[:, None] * col_w[None, :]
    recip = 1.0 / jnp.maximum(area, 1.0)

    xt = jnp.transpose(input, (0, 1, 3, 2))
    mexp = _expanded_rows_tc(xt, mcol, mcolt, recip, row_id)
    out_t = _broadcast_sc(mexp, row_id)
    return jnp.transpose(out_t, (0, 1, 3, 2))
